# baseline (speedup 1.0000x reference)
import functools

import jax
import jax.numpy as jnp
from jax import lax
from jax.experimental import pallas as pl
from jax.experimental.pallas import tpu as pltpu
from jax.experimental.pallas import tpu_sc as plsc

_N = 10000
_E = 320000
_D = 128
_U = 16
_H = 8
_OUT = 7

_NP = 10240
_EPAD = 327680
_SB = 16
_B = 128
_EPW = _EPAD // 16
_NBLK = _EPW // _B
_RPS = _NP // 16
_RB = 1024


def _permute(v, idx_const):
    return lax.gather(
        v, idx_const[:, None],
        lax.GatherDimensionNumbers(offset_dims=(), collapsed_slice_dims=(0,),
                                   start_index_map=(0,)),
        (1,), mode=lax.GatherScatterMode.PROMISE_IN_BOUNDS)


def _splat(v, h):
    return _permute(v, jnp.full((16,), h, jnp.int32))


def _edge_sc(xtp2, sdp2, src2, dst2, dstr):
    mesh = plsc.VectorSubcoreMesh(core_axis_name="c", subcore_axis_name="s")

    @functools.partial(
        pl.kernel, mesh=mesh,
        compiler_params=pltpu.CompilerParams(use_tc_tiling_on_sc=False,
                                             needs_layout_passes=False),
        out_type=jax.ShapeDtypeStruct((2, _NP, 80), jnp.float32),
        scratch_types=[
            pltpu.VMEM((_SB, _B), jnp.int32),
            pltpu.VMEM((_SB, _B), jnp.int32),
            pltpu.VMEM((_SB, _B), jnp.int32),
            [pltpu.VMEM((_B, 80), jnp.float32) for _ in range(2)],
            [pltpu.VMEM((_B, 16), jnp.float32) for _ in range(2)],
            [pltpu.VMEM((_B, 80), jnp.float32) for _ in range(2)],
            pltpu.VMEM((_B, 80), jnp.float32),
            pltpu.VMEM_SHARED((_NP, 80), jnp.float32),
            [pltpu.SemaphoreType.DMA for _ in range(2)],
        ],
    )
    def k(xtp_h, sdp_h, src_h, dst_h, dstr_h, out_h,
          srcv, dstg, dstv, xg, sg, cb, zb, acc, ssems):
        cid = lax.axis_index("c")
        sid = lax.axis_index("s")

        zeros16 = jnp.zeros((16,), jnp.float32)

        def zrow(i, carry):
            for j in range(5):
                zb[i, pl.ds(16 * j, 16)] = zeros16
            for b in range(2):
                cb[b][i, pl.ds(64, 16)] = zeros16
            return carry
        lax.fori_loop(0, _B, zrow, 0)

        for t in range(_RPS // _B):
            pltpu.sync_copy(zb, acc.at[pl.ds(sid * _RPS + t * _B, _B)])
        plsc.subcore_barrier()

        def do_block(tt):
            b = tt % 2
            if tt >= 2:
                pltpu.make_async_copy(cb[b], acc.at[dstv.at[tt]], ssems[b]).wait()
            pltpu.sync_copy(xtp_h.at[srcv.at[tt]], xg[b])
            pltpu.sync_copy(sdp_h.at[dstg.at[tt]], sg[b])

            def edge(e4, c2):
                for u in range(4):
                    e = e4 * 4 + u
                    s = xg[b][e, pl.ds(64, 16)] + sg[b][e, pl.ds(0, 16)]
                    s = jnp.maximum(s, 0.2 * s)
                    s = jnp.minimum(jnp.maximum(s, -2.0), 2.0)
                    w = jnp.exp(s)
                    cb[b][e, pl.ds(64, 16)] = w
                    for h in range(4):
                        cb[b][e, pl.ds(16 * h, 16)] = (
                            _splat(w, h) * xg[b][e, pl.ds(16 * h, 16)])
                return c2
            lax.fori_loop(0, _B // 4, edge, 0)

            pltpu.async_copy(cb[b], acc.at[dstv.at[tt]], ssems[b], add=True)

        def super_block(sb, first_sb):
            if not first_sb:
                for b in range(2):
                    pltpu.make_async_copy(cb[b], acc.at[dstv.at[b]],
                                          ssems[b]).wait()
            base = sb * _SB
            pltpu.sync_copy(src_h.at[cid, sid, pl.ds(base, _SB)], srcv)
            pltpu.sync_copy(dst_h.at[cid, sid, pl.ds(base, _SB)], dstg)
            pltpu.sync_copy(dstr_h.at[sid, pl.ds(base, _SB)], dstv)
            for tt in range(_SB):
                do_block(tt)

        super_block(0, True)

        def sblk(sb, carry):
            super_block(sb, False)
            return carry
        lax.fori_loop(1, _NBLK // _SB, sblk, 0)

        for b in range(2):
            pltpu.make_async_copy(cb[b], acc.at[dstv.at[b]], ssems[b]).wait()

        plsc.subcore_barrier()
        pltpu.sync_copy(acc.at[pl.ds(sid * _RPS, _RPS)],
                        out_h.at[cid, pl.ds(sid * _RPS, _RPS)])

    return k(xtp2, sdp2, src2, dst2, dstr)


def _combine(pa, pb, xprev):
    cols = []
    for h in range(_H):
        p = pa if h < 4 else pb
        hh = h % 4
        den = jnp.maximum(p[:, 64 + hh:65 + hh], 1e-30)
        cols.append(p[:, 16 * hh:16 * (hh + 1)] / den)
    hm = jnp.concatenate(cols, axis=1)
    return jnp.maximum(hm, 0.0) + xprev


def _tc_pre(nsp, W_pre, b_pre, Wa, Wb):
    def body(ns_ref, wp_ref, bp_ref, wa_ref, wb_ref, x0_ref, xtp_ref, sdp_ref):
        x0 = jnp.maximum(
            jnp.dot(ns_ref[...], wp_ref[...], preferred_element_type=jnp.float32)
            + bp_ref[...], 0.0)
        x0_ref[...] = x0
        xtp_ref[...] = jnp.dot(x0, wa_ref[...], preferred_element_type=jnp.float32)
        sdp_ref[...] = jnp.dot(x0, wb_ref[...], preferred_element_type=jnp.float32)

    return pl.pallas_call(
        body,
        grid=(_NP // _RB,),
        in_specs=[
            pl.BlockSpec((_RB, _D), lambda i: (i, 0)),
            pl.BlockSpec((_D, _D), lambda i: (0, 0)),
            pl.BlockSpec((1, _D), lambda i: (0, 0)),
            pl.BlockSpec((_D, 160), lambda i: (0, 0)),
            pl.BlockSpec((_D, 32), lambda i: (0, 0)),
        ],
        out_specs=[
            pl.BlockSpec((_RB, _D), lambda i: (i, 0)),
            pl.BlockSpec((_RB, 160), lambda i: (i, 0)),
            pl.BlockSpec((_RB, 32), lambda i: (i, 0)),
        ],
        out_shape=[
            jax.ShapeDtypeStruct((_NP, _D), jnp.float32),
            jax.ShapeDtypeStruct((_NP, 160), jnp.float32),
            jax.ShapeDtypeStruct((_NP, 32), jnp.float32),
        ],
    )(nsp, W_pre, b_pre.reshape(1, _D), Wa, Wb)


def _tc_mid(pa, pb, xprev, Wa, Wb):
    def body(pa_ref, pb_ref, xp_ref, wa_ref, wb_ref, x1_ref, xtp_ref, sdp_ref):
        x1 = _combine(pa_ref[...], pb_ref[...], xp_ref[...])
        x1_ref[...] = x1
        xtp_ref[...] = jnp.dot(x1, wa_ref[...], preferred_element_type=jnp.float32)
        sdp_ref[...] = jnp.dot(x1, wb_ref[...], preferred_element_type=jnp.float32)

    return pl.pallas_call(
        body,
        grid=(_NP // _RB,),
        in_specs=[
            pl.BlockSpec((_RB, 80), lambda i: (i, 0)),
            pl.BlockSpec((_RB, 80), lambda i: (i, 0)),
            pl.BlockSpec((_RB, _D), lambda i: (i, 0)),
            pl.BlockSpec((_D, 160), lambda i: (0, 0)),
            pl.BlockSpec((_D, 32), lambda i: (0, 0)),
        ],
        out_specs=[
            pl.BlockSpec((_RB, _D), lambda i: (i, 0)),
            pl.BlockSpec((_RB, 160), lambda i: (i, 0)),
            pl.BlockSpec((_RB, 32), lambda i: (i, 0)),
        ],
        out_shape=[
            jax.ShapeDtypeStruct((_NP, _D), jnp.float32),
            jax.ShapeDtypeStruct((_NP, 160), jnp.float32),
            jax.ShapeDtypeStruct((_NP, 32), jnp.float32),
        ],
    )(pa, pb, xprev, Wa, Wb)


def _tc_out(pa, pb, xprev, Wo, bo):
    def body(pa_ref, pb_ref, xp_ref, wo_ref, bo_ref, out_ref):
        x2 = _combine(pa_ref[...], pb_ref[...], xp_ref[...])
        out_ref[...] = jnp.dot(x2, wo_ref[...],
                               preferred_element_type=jnp.float32) + bo_ref[...]

    return pl.pallas_call(
        body,
        grid=(_NP // _RB,),
        in_specs=[
            pl.BlockSpec((_RB, 80), lambda i: (i, 0)),
            pl.BlockSpec((_RB, 80), lambda i: (i, 0)),
            pl.BlockSpec((_RB, _D), lambda i: (i, 0)),
            pl.BlockSpec((_D, 8), lambda i: (0, 0)),
            pl.BlockSpec((1, 8), lambda i: (0, 0)),
        ],
        out_specs=[pl.BlockSpec((_RB, 8), lambda i: (i, 0))],
        out_shape=[jax.ShapeDtypeStruct((_NP, 8), jnp.float32)],
    )(pa, pb, xprev, Wo, bo)[0]


def _layer_weights(kernels, atts, l):
    K = kernels[l].transpose(1, 0, 2).reshape(_D, _U * _H)
    Kh = K.reshape(_D, _H, _U)
    As = jnp.stack([atts[l, h, _U:, 0] for h in range(_H)], 1)
    Ad = jnp.stack([atts[l, h, :_U, 0] for h in range(_H)], 1)
    Scol = jnp.einsum("dhu,uh->dh", Kh, As)
    Dcol = jnp.einsum("dhu,uh->dh", Kh, Ad)
    z12 = jnp.zeros((_D, 12), jnp.float32)
    wa_halves, wb_halves = [], []
    for p in range(2):
        wa_halves += [K[:, 64 * p:64 * (p + 1)], Scol[:, 4 * p:4 * (p + 1)], z12]
        wb_halves += [Dcol[:, 4 * p:4 * (p + 1)], z12]
    Wa = jnp.concatenate(wa_halves, 1)
    Wb = jnp.concatenate(wb_halves, 1)
    return Wa, Wb


def _split_halves(xtp, sdp):
    xtp2 = jnp.concatenate([xtp[:, :80], xtp[:, 80:]], 0)
    sdp2 = jnp.concatenate([sdp[:, :16], sdp[:, 16:]], 0)
    return xtp2, sdp2


def kernel(node_states, edges, W_pre, b_pre, kernels, atts, W_out, b_out):
    e32 = edges.astype(jnp.int32)
    dst = jnp.concatenate([e32[:, 0], jnp.full((_EPAD - _E,), _N, jnp.int32)])
    src = jnp.concatenate([e32[:, 1], jnp.full((_EPAD - _E,), _N, jnp.int32)])
    nsp = jnp.pad(node_states, ((0, _NP - _N), (0, 0)))

    Wa0, Wb0 = _layer_weights(kernels, atts, 0)
    Wa1, Wb1 = _layer_weights(kernels, atts, 1)
    Wo = jnp.pad(W_out, ((0, 0), (0, 1)))
    bo = jnp.pad(b_out, (0, 1)).reshape(1, 8)

    src2 = jnp.stack([src, src + _NP]).reshape(2, 16, _NBLK, _B)
    dst2 = jnp.stack([dst, dst + _NP]).reshape(2, 16, _NBLK, _B)
    dstr = dst.reshape(16, _NBLK, _B)

    x0, xtp0, sdp0 = _tc_pre(nsp, W_pre, b_pre, Wa0, Wb0)
    p = _edge_sc(*_split_halves(xtp0, sdp0), src2, dst2, dstr)
    x1, xtp1, sdp1 = _tc_mid(p[0], p[1], x0, Wa1, Wb1)
    q = _edge_sc(*_split_halves(xtp1, sdp1), src2, dst2, dstr)
    out = _tc_out(q[0], q[1], x1, Wo, bo)
    return out[:_N, :_OUT]


# speedup vs baseline: 1.5552x; 1.5552x over previous
import functools

import jax
import jax.numpy as jnp
from jax import lax
from jax.experimental import pallas as pl
from jax.experimental.pallas import tpu as pltpu
from jax.experimental.pallas import tpu_sc as plsc

_N = 10000
_E = 320000
_D = 128
_U = 16
_H = 8
_OUT = 7

_NP = 10240
_EPAD = 327680
_SB = 16
_B = 128
_EPW = _EPAD // 16
_NBLK = _EPW // _B
_RPS = _NP // 16
_RB = 1024


def _permute(v, idx_const):
    return lax.gather(
        v, idx_const[:, None],
        lax.GatherDimensionNumbers(offset_dims=(), collapsed_slice_dims=(0,),
                                   start_index_map=(0,)),
        (1,), mode=lax.GatherScatterMode.PROMISE_IN_BOUNDS)


def _splat(v, h):
    return _permute(v, jnp.full((16,), h, jnp.int32))


def _edge_sc(xtp2, sdp2, src2, dst2, dstr):
    mesh = plsc.VectorSubcoreMesh(core_axis_name="c", subcore_axis_name="s")

    @functools.partial(
        pl.kernel, mesh=mesh,
        compiler_params=pltpu.CompilerParams(use_tc_tiling_on_sc=False,
                                             needs_layout_passes=False),
        out_type=jax.ShapeDtypeStruct((2, _NP, 80), jnp.float32),
        scratch_types=[
            pltpu.VMEM((2 * _SB, _B), jnp.int32),
            pltpu.VMEM((2 * _SB, _B), jnp.int32),
            pltpu.VMEM((2 * _SB, _B), jnp.int32),
            [pltpu.VMEM((_B, 80), jnp.float32) for _ in range(2)],
            [pltpu.VMEM((_B, 16), jnp.float32) for _ in range(2)],
            [pltpu.VMEM((_B, 80), jnp.float32) for _ in range(2)],
            pltpu.VMEM((_B, 80), jnp.float32),
            pltpu.VMEM_SHARED((_NP, 80), jnp.float32),
            [pltpu.SemaphoreType.DMA for _ in range(2)],
            [pltpu.SemaphoreType.DMA for _ in range(2)],
        ],
    )
    def k(xtp_h, sdp_h, src_h, dst_h, dstr_h, out_h,
          srcv, dstg, dstv, xg, sg, cb, zb, acc, gsems, ssems):
        cid = lax.axis_index("c")
        sid = lax.axis_index("s")

        zeros16 = jnp.zeros((16,), jnp.float32)

        def zrow(i, carry):
            for j in range(5):
                zb[i, pl.ds(16 * j, 16)] = zeros16
            for b in range(2):
                cb[b][i, pl.ds(64, 16)] = zeros16
            return carry
        lax.fori_loop(0, _B, zrow, 0)

        for t in range(_RPS // _B):
            pltpu.sync_copy(zb, acc.at[pl.ds(sid * _RPS + t * _B, _B)])
        plsc.subcore_barrier()

        _NSB = _NBLK // _SB

        def load_idx(sb, u):
            base = sb * _SB
            half = pl.ds(u * _SB, _SB)
            pltpu.sync_copy(src_h.at[cid, sid, pl.ds(base, _SB)], srcv.at[half])
            pltpu.sync_copy(dst_h.at[cid, sid, pl.ds(base, _SB)], dstg.at[half])
            pltpu.sync_copy(dstr_h.at[sid, pl.ds(base, _SB)], dstv.at[half])

        def issue_gather(row, b):
            pltpu.async_copy(xtp_h.at[srcv.at[row]], xg[b], gsems[b])
            pltpu.async_copy(sdp_h.at[dstg.at[row]], sg[b], gsems[b])

        def wait_gather(row, b):
            pltpu.make_async_copy(xtp_h.at[srcv.at[row]], xg[b], gsems[b]).wait()
            pltpu.make_async_copy(sdp_h.at[dstg.at[row]], sg[b], gsems[b]).wait()

        def do_block(tt, u, sb):
            b = tt % 2
            row = u * _SB + tt
            if tt >= 2:
                pltpu.make_async_copy(cb[b], acc.at[dstv.at[row]], ssems[b]).wait()
            wait_gather(row, b)

            def edge(e4, c2):
                for uu in range(4):
                    e = e4 * 4 + uu
                    s = xg[b][e, pl.ds(64, 16)] + sg[b][e, pl.ds(0, 16)]
                    s = jnp.maximum(s, 0.2 * s)
                    s = jnp.minimum(jnp.maximum(s, -2.0), 2.0)
                    w = jnp.exp(s)
                    cb[b][e, pl.ds(64, 16)] = w
                    for h in range(4):
                        cb[b][e, pl.ds(16 * h, 16)] = (
                            _splat(w, h) * xg[b][e, pl.ds(16 * h, 16)])
                return c2
            lax.fori_loop(0, _B // 4, edge, 0)

            pltpu.async_copy(cb[b], acc.at[dstv.at[row]], ssems[b], add=True)

            if tt + 2 < _SB:
                issue_gather(row + 2, b)
            else:
                nrow = (1 - u) * _SB + (tt + 2 - _SB)

                @pl.when(sb < _NSB - 1)
                def _():
                    issue_gather(nrow, b)

        def super_block(sb, u, first_sb):
            if not first_sb:
                for b in range(2):
                    pltpu.make_async_copy(cb[b], acc.at[dstv.at[b]],
                                          ssems[b]).wait()

            @pl.when(sb < _NSB - 1)
            def _():
                load_idx(sb + 1, 1 - u)
            for tt in range(_SB):
                do_block(tt, u, sb)

        load_idx(0, 0)
        issue_gather(0, 0)
        issue_gather(1, 1)
        super_block(0, 0, True)

        def sblk(sb, carry):
            super_block(sb, lax.rem(sb, 2), False)
            return carry
        lax.fori_loop(1, _NSB, sblk, 0)

        for b in range(2):
            pltpu.make_async_copy(cb[b], acc.at[dstv.at[b]], ssems[b]).wait()

        plsc.subcore_barrier()
        pltpu.sync_copy(acc.at[pl.ds(sid * _RPS, _RPS)],
                        out_h.at[cid, pl.ds(sid * _RPS, _RPS)])

    return k(xtp2, sdp2, src2, dst2, dstr)


def _combine(pa, pb, xprev):
    cols = []
    for h in range(_H):
        p = pa if h < 4 else pb
        hh = h % 4
        den = jnp.maximum(p[:, 64 + hh:65 + hh], 1e-30)
        cols.append(p[:, 16 * hh:16 * (hh + 1)] / den)
    hm = jnp.concatenate(cols, axis=1)
    return jnp.maximum(hm, 0.0) + xprev


def _tc_pre(nsp, W_pre, b_pre, Wa, Wb):
    def body(ns_ref, wp_ref, bp_ref, wa_ref, wb_ref, x0_ref, xtp_ref, sdp_ref):
        x0 = jnp.maximum(
            jnp.dot(ns_ref[...], wp_ref[...], preferred_element_type=jnp.float32)
            + bp_ref[...], 0.0)
        x0_ref[...] = x0
        xtp_ref[...] = jnp.dot(x0, wa_ref[...], preferred_element_type=jnp.float32)
        sdp_ref[...] = jnp.dot(x0, wb_ref[...], preferred_element_type=jnp.float32)

    return pl.pallas_call(
        body,
        grid=(_NP // _RB,),
        in_specs=[
            pl.BlockSpec((_RB, _D), lambda i: (i, 0)),
            pl.BlockSpec((_D, _D), lambda i: (0, 0)),
            pl.BlockSpec((1, _D), lambda i: (0, 0)),
            pl.BlockSpec((_D, 160), lambda i: (0, 0)),
            pl.BlockSpec((_D, 32), lambda i: (0, 0)),
        ],
        out_specs=[
            pl.BlockSpec((_RB, _D), lambda i: (i, 0)),
            pl.BlockSpec((_RB, 160), lambda i: (i, 0)),
            pl.BlockSpec((_RB, 32), lambda i: (i, 0)),
        ],
        out_shape=[
            jax.ShapeDtypeStruct((_NP, _D), jnp.float32),
            jax.ShapeDtypeStruct((_NP, 160), jnp.float32),
            jax.ShapeDtypeStruct((_NP, 32), jnp.float32),
        ],
    )(nsp, W_pre, b_pre.reshape(1, _D), Wa, Wb)


def _tc_mid(pa, pb, xprev, Wa, Wb):
    def body(pa_ref, pb_ref, xp_ref, wa_ref, wb_ref, x1_ref, xtp_ref, sdp_ref):
        x1 = _combine(pa_ref[...], pb_ref[...], xp_ref[...])
        x1_ref[...] = x1
        xtp_ref[...] = jnp.dot(x1, wa_ref[...], preferred_element_type=jnp.float32)
        sdp_ref[...] = jnp.dot(x1, wb_ref[...], preferred_element_type=jnp.float32)

    return pl.pallas_call(
        body,
        grid=(_NP // _RB,),
        in_specs=[
            pl.BlockSpec((_RB, 80), lambda i: (i, 0)),
            pl.BlockSpec((_RB, 80), lambda i: (i, 0)),
            pl.BlockSpec((_RB, _D), lambda i: (i, 0)),
            pl.BlockSpec((_D, 160), lambda i: (0, 0)),
            pl.BlockSpec((_D, 32), lambda i: (0, 0)),
        ],
        out_specs=[
            pl.BlockSpec((_RB, _D), lambda i: (i, 0)),
            pl.BlockSpec((_RB, 160), lambda i: (i, 0)),
            pl.BlockSpec((_RB, 32), lambda i: (i, 0)),
        ],
        out_shape=[
            jax.ShapeDtypeStruct((_NP, _D), jnp.float32),
            jax.ShapeDtypeStruct((_NP, 160), jnp.float32),
            jax.ShapeDtypeStruct((_NP, 32), jnp.float32),
        ],
    )(pa, pb, xprev, Wa, Wb)


def _tc_out(pa, pb, xprev, Wo, bo):
    def body(pa_ref, pb_ref, xp_ref, wo_ref, bo_ref, out_ref):
        x2 = _combine(pa_ref[...], pb_ref[...], xp_ref[...])
        out_ref[...] = jnp.dot(x2, wo_ref[...],
                               preferred_element_type=jnp.float32) + bo_ref[...]

    return pl.pallas_call(
        body,
        grid=(_NP // _RB,),
        in_specs=[
            pl.BlockSpec((_RB, 80), lambda i: (i, 0)),
            pl.BlockSpec((_RB, 80), lambda i: (i, 0)),
            pl.BlockSpec((_RB, _D), lambda i: (i, 0)),
            pl.BlockSpec((_D, 8), lambda i: (0, 0)),
            pl.BlockSpec((1, 8), lambda i: (0, 0)),
        ],
        out_specs=[pl.BlockSpec((_RB, 8), lambda i: (i, 0))],
        out_shape=[jax.ShapeDtypeStruct((_NP, 8), jnp.float32)],
    )(pa, pb, xprev, Wo, bo)[0]


def _layer_weights(kernels, atts, l):
    K = kernels[l].transpose(1, 0, 2).reshape(_D, _U * _H)
    Kh = K.reshape(_D, _H, _U)
    As = jnp.stack([atts[l, h, _U:, 0] for h in range(_H)], 1)
    Ad = jnp.stack([atts[l, h, :_U, 0] for h in range(_H)], 1)
    Scol = jnp.einsum("dhu,uh->dh", Kh, As)
    Dcol = jnp.einsum("dhu,uh->dh", Kh, Ad)
    z12 = jnp.zeros((_D, 12), jnp.float32)
    wa_halves, wb_halves = [], []
    for p in range(2):
        wa_halves += [K[:, 64 * p:64 * (p + 1)], Scol[:, 4 * p:4 * (p + 1)], z12]
        wb_halves += [Dcol[:, 4 * p:4 * (p + 1)], z12]
    Wa = jnp.concatenate(wa_halves, 1)
    Wb = jnp.concatenate(wb_halves, 1)
    return Wa, Wb


def _split_halves(xtp, sdp):
    xtp2 = jnp.concatenate([xtp[:, :80], xtp[:, 80:]], 0)
    sdp2 = jnp.concatenate([sdp[:, :16], sdp[:, 16:]], 0)
    return xtp2, sdp2


def kernel(node_states, edges, W_pre, b_pre, kernels, atts, W_out, b_out):
    e32 = edges.astype(jnp.int32)
    dst = jnp.concatenate([e32[:, 0], jnp.full((_EPAD - _E,), _N, jnp.int32)])
    src = jnp.concatenate([e32[:, 1], jnp.full((_EPAD - _E,), _N, jnp.int32)])
    nsp = jnp.pad(node_states, ((0, _NP - _N), (0, 0)))

    Wa0, Wb0 = _layer_weights(kernels, atts, 0)
    Wa1, Wb1 = _layer_weights(kernels, atts, 1)
    Wo = jnp.pad(W_out, ((0, 0), (0, 1)))
    bo = jnp.pad(b_out, (0, 1)).reshape(1, 8)

    src2 = jnp.stack([src, src + _NP]).reshape(2, 16, _NBLK, _B)
    dst2 = jnp.stack([dst, dst + _NP]).reshape(2, 16, _NBLK, _B)
    dstr = dst.reshape(16, _NBLK, _B)

    x0, xtp0, sdp0 = _tc_pre(nsp, W_pre, b_pre, Wa0, Wb0)
    p = _edge_sc(*_split_halves(xtp0, sdp0), src2, dst2, dstr)
    x1, xtp1, sdp1 = _tc_mid(p[0], p[1], x0, Wa1, Wb1)
    q = _edge_sc(*_split_halves(xtp1, sdp1), src2, dst2, dstr)
    out = _tc_out(q[0], q[1], x1, Wo, bo)
    return out[:_N, :_OUT]


# speedup vs baseline: 1.9722x; 1.2681x over previous
import functools

import jax
import jax.numpy as jnp
from jax import lax
from jax.experimental import pallas as pl
from jax.experimental.pallas import tpu as pltpu
from jax.experimental.pallas import tpu_sc as plsc

_N = 10000
_E = 320000
_D = 128
_U = 16
_H = 8
_OUT = 7

_NP = 10240
_EPAD = 327680
_SB = 16
_B = 128
_EPW = _EPAD // 16
_NBLK = _EPW // _B
_RPS = _NP // 16
_RB = 1024


def _permute(v, idx_const):
    return lax.gather(
        v, idx_const[:, None],
        lax.GatherDimensionNumbers(offset_dims=(), collapsed_slice_dims=(0,),
                                   start_index_map=(0,)),
        (1,), mode=lax.GatherScatterMode.PROMISE_IN_BOUNDS)


def _splat(v, h):
    return _permute(v, jnp.full((16,), h, jnp.int32))


def _edge_sc(xtp2, sdp2, src2, dst2, dstr):
    mesh = plsc.VectorSubcoreMesh(core_axis_name="c", subcore_axis_name="s")

    @functools.partial(
        pl.kernel, mesh=mesh,
        compiler_params=pltpu.CompilerParams(use_tc_tiling_on_sc=False,
                                             needs_layout_passes=False),
        out_type=jax.ShapeDtypeStruct((2, _NP, 80), jnp.float32),
        scratch_types=[
            pltpu.VMEM((2 * _SB, _B), jnp.int32),
            pltpu.VMEM((2 * _SB, _B), jnp.int32),
            pltpu.VMEM((2 * _SB, _B), jnp.int32),
            [pltpu.VMEM((_B, 80), jnp.float32) for _ in range(2)],
            [pltpu.VMEM((_B, 16), jnp.float32) for _ in range(2)],
            [pltpu.VMEM((_B, 80), jnp.float32) for _ in range(2)],
            pltpu.VMEM((_B, 80), jnp.float32),
            pltpu.VMEM_SHARED((_NP, 80), jnp.float32),
            [pltpu.SemaphoreType.DMA for _ in range(2)],
            [pltpu.SemaphoreType.DMA for _ in range(2)],
        ],
    )
    def k(xtp_h, sdp_h, src_h, dst_h, dstr_h, out_h,
          srcv, dstg, dstv, xg, sg, cb, zb, acc, gsems, ssems):
        cid = lax.axis_index("c")
        sid = lax.axis_index("s")

        zeros16 = jnp.zeros((16,), jnp.float32)

        def zrow(i, carry):
            for j in range(5):
                zb[i, pl.ds(16 * j, 16)] = zeros16
            for b in range(2):
                cb[b][i, pl.ds(64, 16)] = zeros16
            return carry
        lax.fori_loop(0, _B, zrow, 0)

        for t in range(_RPS // _B):
            pltpu.sync_copy(zb, acc.at[pl.ds(sid * _RPS + t * _B, _B)])
        plsc.subcore_barrier()

        _NSB = _NBLK // _SB

        def load_idx(sb, u):
            base = sb * _SB
            half = pl.ds(u * _SB, _SB)
            pltpu.sync_copy(src_h.at[cid, sid, pl.ds(base, _SB)], srcv.at[half])
            pltpu.sync_copy(dst_h.at[cid, sid, pl.ds(base, _SB)], dstg.at[half])
            pltpu.sync_copy(dstr_h.at[sid, pl.ds(base, _SB)], dstv.at[half])

        def issue_gather(row, b):
            pltpu.async_copy(xtp_h.at[srcv.at[row]], xg[b], gsems[b])
            pltpu.async_copy(sdp_h.at[dstg.at[row]], sg[b], gsems[b])

        def wait_gather(row, b):
            pltpu.make_async_copy(xtp_h.at[srcv.at[row]], xg[b], gsems[b]).wait()
            pltpu.make_async_copy(sdp_h.at[dstg.at[row]], sg[b], gsems[b]).wait()

        def do_block(tt, b, u, wait, prefetch_row, prefetch_cond):
            row = u * _SB + tt
            if wait:
                pltpu.make_async_copy(cb[b], acc.at[dstv.at[row]], ssems[b]).wait()
            wait_gather(row, b)

            def edge8(e8, c2):
                e0 = e8 * 8
                ws = []
                for uu in range(8):
                    e = e0 + uu
                    s = xg[b][e, pl.ds(64, 16)] + sg[b][e, pl.ds(0, 16)]
                    s = jnp.maximum(s, 0.2 * s)
                    s = jnp.minimum(jnp.maximum(s, -2.0), 2.0)
                    ws.append(jnp.exp(s))
                for uu in range(8):
                    e = e0 + uu
                    w = ws[uu]
                    cb[b][e, pl.ds(64, 16)] = w
                    for h in range(4):
                        cb[b][e, pl.ds(16 * h, 16)] = (
                            _splat(w, h) * xg[b][e, pl.ds(16 * h, 16)])
                return c2
            lax.fori_loop(0, _B // 8, edge8, 0)

            pltpu.async_copy(cb[b], acc.at[dstv.at[row]], ssems[b], add=True)

            if prefetch_cond is None or prefetch_cond is True:
                issue_gather(prefetch_row, b)
            else:
                @pl.when(prefetch_cond)
                def _():
                    issue_gather(prefetch_row, b)

        def super_block(sb, u, first_sb):
            if not first_sb:
                for b in range(2):
                    pltpu.make_async_copy(cb[b], acc.at[dstv.at[b]],
                                          ssems[b]).wait()

            @pl.when(sb < _NSB - 1)
            def _():
                load_idx(sb + 1, 1 - u)

            for b in range(2):
                do_block(b, b, u, False, u * _SB + b + 2, None)

            def pair(kk, c2):
                tt0 = 2 * kk
                for b in range(2):
                    do_block(tt0 + b, b, u, True, u * _SB + tt0 + b + 2, None)
                return c2
            lax.fori_loop(1, 7, pair, 0)

            for b in range(2):
                do_block(14 + b, b, u, True, (1 - u) * _SB + b,
                         (sb < _NSB - 1))

        load_idx(0, 0)
        issue_gather(0, 0)
        issue_gather(1, 1)
        super_block(0, 0, True)

        def sblk(sb, carry):
            super_block(sb, lax.rem(sb, 2), False)
            return carry
        lax.fori_loop(1, _NSB, sblk, 0)

        for b in range(2):
            pltpu.make_async_copy(cb[b], acc.at[dstv.at[b]], ssems[b]).wait()

        plsc.subcore_barrier()
        pltpu.sync_copy(acc.at[pl.ds(sid * _RPS, _RPS)],
                        out_h.at[cid, pl.ds(sid * _RPS, _RPS)])

    return k(xtp2, sdp2, src2, dst2, dstr)


def _combine(pa, pb, xprev):
    cols = []
    for h in range(_H):
        p = pa if h < 4 else pb
        hh = h % 4
        den = jnp.maximum(p[:, 64 + hh:65 + hh], 1e-30)
        cols.append(p[:, 16 * hh:16 * (hh + 1)] / den)
    hm = jnp.concatenate(cols, axis=1)
    return jnp.maximum(hm, 0.0) + xprev


def _tc_pre(nsp, W_pre, b_pre, Wa, Wb):
    def body(ns_ref, wp_ref, bp_ref, wa_ref, wb_ref, x0_ref, xtp_ref, sdp_ref):
        x0 = jnp.maximum(
            jnp.dot(ns_ref[...], wp_ref[...], preferred_element_type=jnp.float32)
            + bp_ref[...], 0.0)
        x0_ref[...] = x0
        xtp_ref[...] = jnp.dot(x0, wa_ref[...], preferred_element_type=jnp.float32)
        sdp_ref[...] = jnp.dot(x0, wb_ref[...], preferred_element_type=jnp.float32)

    return pl.pallas_call(
        body,
        grid=(_NP // _RB,),
        in_specs=[
            pl.BlockSpec((_RB, _D), lambda i: (i, 0)),
            pl.BlockSpec((_D, _D), lambda i: (0, 0)),
            pl.BlockSpec((1, _D), lambda i: (0, 0)),
            pl.BlockSpec((_D, 160), lambda i: (0, 0)),
            pl.BlockSpec((_D, 32), lambda i: (0, 0)),
        ],
        out_specs=[
            pl.BlockSpec((_RB, _D), lambda i: (i, 0)),
            pl.BlockSpec((_RB, 160), lambda i: (i, 0)),
            pl.BlockSpec((_RB, 32), lambda i: (i, 0)),
        ],
        out_shape=[
            jax.ShapeDtypeStruct((_NP, _D), jnp.float32),
            jax.ShapeDtypeStruct((_NP, 160), jnp.float32),
            jax.ShapeDtypeStruct((_NP, 32), jnp.float32),
        ],
    )(nsp, W_pre, b_pre.reshape(1, _D), Wa, Wb)


def _tc_mid(pa, pb, xprev, Wa, Wb):
    def body(pa_ref, pb_ref, xp_ref, wa_ref, wb_ref, x1_ref, xtp_ref, sdp_ref):
        x1 = _combine(pa_ref[...], pb_ref[...], xp_ref[...])
        x1_ref[...] = x1
        xtp_ref[...] = jnp.dot(x1, wa_ref[...], preferred_element_type=jnp.float32)
        sdp_ref[...] = jnp.dot(x1, wb_ref[...], preferred_element_type=jnp.float32)

    return pl.pallas_call(
        body,
        grid=(_NP // _RB,),
        in_specs=[
            pl.BlockSpec((_RB, 80), lambda i: (i, 0)),
            pl.BlockSpec((_RB, 80), lambda i: (i, 0)),
            pl.BlockSpec((_RB, _D), lambda i: (i, 0)),
            pl.BlockSpec((_D, 160), lambda i: (0, 0)),
            pl.BlockSpec((_D, 32), lambda i: (0, 0)),
        ],
        out_specs=[
            pl.BlockSpec((_RB, _D), lambda i: (i, 0)),
            pl.BlockSpec((_RB, 160), lambda i: (i, 0)),
            pl.BlockSpec((_RB, 32), lambda i: (i, 0)),
        ],
        out_shape=[
            jax.ShapeDtypeStruct((_NP, _D), jnp.float32),
            jax.ShapeDtypeStruct((_NP, 160), jnp.float32),
            jax.ShapeDtypeStruct((_NP, 32), jnp.float32),
        ],
    )(pa, pb, xprev, Wa, Wb)


def _tc_out(pa, pb, xprev, Wo, bo):
    def body(pa_ref, pb_ref, xp_ref, wo_ref, bo_ref, out_ref):
        x2 = _combine(pa_ref[...], pb_ref[...], xp_ref[...])
        out_ref[...] = jnp.dot(x2, wo_ref[...],
                               preferred_element_type=jnp.float32) + bo_ref[...]

    return pl.pallas_call(
        body,
        grid=(_NP // _RB,),
        in_specs=[
            pl.BlockSpec((_RB, 80), lambda i: (i, 0)),
            pl.BlockSpec((_RB, 80), lambda i: (i, 0)),
            pl.BlockSpec((_RB, _D), lambda i: (i, 0)),
            pl.BlockSpec((_D, 8), lambda i: (0, 0)),
            pl.BlockSpec((1, 8), lambda i: (0, 0)),
        ],
        out_specs=[pl.BlockSpec((_RB, 8), lambda i: (i, 0))],
        out_shape=[jax.ShapeDtypeStruct((_NP, 8), jnp.float32)],
    )(pa, pb, xprev, Wo, bo)[0]


def _layer_weights(kernels, atts, l):
    K = kernels[l].transpose(1, 0, 2).reshape(_D, _U * _H)
    Kh = K.reshape(_D, _H, _U)
    As = jnp.stack([atts[l, h, _U:, 0] for h in range(_H)], 1)
    Ad = jnp.stack([atts[l, h, :_U, 0] for h in range(_H)], 1)
    Scol = jnp.einsum("dhu,uh->dh", Kh, As)
    Dcol = jnp.einsum("dhu,uh->dh", Kh, Ad)
    z12 = jnp.zeros((_D, 12), jnp.float32)
    wa_halves, wb_halves = [], []
    for p in range(2):
        wa_halves += [K[:, 64 * p:64 * (p + 1)], Scol[:, 4 * p:4 * (p + 1)], z12]
        wb_halves += [Dcol[:, 4 * p:4 * (p + 1)], z12]
    Wa = jnp.concatenate(wa_halves, 1)
    Wb = jnp.concatenate(wb_halves, 1)
    return Wa, Wb


def _split_halves(xtp, sdp):
    xtp2 = jnp.concatenate([xtp[:, :80], xtp[:, 80:]], 0)
    sdp2 = jnp.concatenate([sdp[:, :16], sdp[:, 16:]], 0)
    return xtp2, sdp2


def kernel(node_states, edges, W_pre, b_pre, kernels, atts, W_out, b_out):
    e32 = edges.astype(jnp.int32)
    dst = jnp.concatenate([e32[:, 0], jnp.full((_EPAD - _E,), _N, jnp.int32)])
    src = jnp.concatenate([e32[:, 1], jnp.full((_EPAD - _E,), _N, jnp.int32)])
    nsp = jnp.pad(node_states, ((0, _NP - _N), (0, 0)))

    Wa0, Wb0 = _layer_weights(kernels, atts, 0)
    Wa1, Wb1 = _layer_weights(kernels, atts, 1)
    Wo = jnp.pad(W_out, ((0, 0), (0, 1)))
    bo = jnp.pad(b_out, (0, 1)).reshape(1, 8)

    src2 = jnp.stack([src, src + _NP]).reshape(2, 16, _NBLK, _B)
    dst2 = jnp.stack([dst, dst + _NP]).reshape(2, 16, _NBLK, _B)
    dstr = dst.reshape(16, _NBLK, _B)

    x0, xtp0, sdp0 = _tc_pre(nsp, W_pre, b_pre, Wa0, Wb0)
    p = _edge_sc(*_split_halves(xtp0, sdp0), src2, dst2, dstr)
    x1, xtp1, sdp1 = _tc_mid(p[0], p[1], x0, Wa1, Wb1)
    q = _edge_sc(*_split_halves(xtp1, sdp1), src2, dst2, dstr)
    out = _tc_out(q[0], q[1], x1, Wo, bo)
    return out[:_N, :_OUT]
